# baseline (speedup 1.0000x reference)
import functools

import jax
import jax.numpy as jnp
from jax import lax
from jax.experimental import pallas as pl
from jax.experimental.pallas import tpu as pltpu
from jax.experimental.pallas import tpu_sc as plsc

_NW = 32
_BQ = 16
_NB = 4


def kernel(x, table):
    bsz, seq = x.shape
    vocab, d = table.shape
    ndt = d // 8
    nbt = bsz // 128
    nq = 128 // _BQ
    nunits = seq * nbt * nq
    upw = nunits // _NW
    units_per_s = nbt * nq

    xt = jnp.transpose(x).reshape(-1).astype(jnp.int32)

    mesh = plsc.VectorSubcoreMesh(core_axis_name="c", subcore_axis_name="s")

    @functools.partial(
        pl.kernel,
        mesh=mesh,
        out_type=jax.ShapeDtypeStruct((seq, ndt, nbt, 8, 128), jnp.float32),
        compiler_params=pltpu.CompilerParams(
            use_tc_tiling_on_sc=False, needs_layout_passes=False
        ),
        scratch_types=[
            [pltpu.VMEM((_BQ, d), jnp.float32) for _ in range(_NB)],
            [pltpu.VMEM((ndt, 8, _BQ), jnp.float32) for _ in range(_NB)],
            pltpu.VMEM((seq * bsz // _NW,), jnp.int32),
            [pltpu.SemaphoreType.DMA for _ in range(_NB)],
            [pltpu.SemaphoreType.DMA for _ in range(_NB)],
        ],
    )
    def k(table_hbm, xt_hbm, out_hbm, srcs, dsts, idx_all, sem_g, sem_w):
        wid = lax.axis_index("s") * 2 + lax.axis_index("c")
        u0 = wid * upw
        pltpu.sync_copy(xt_hbm.at[pl.ds(u0 * _BQ, upw * _BQ)], idx_all)

        def decode(u):
            s = u // units_per_s
            r = u - s * units_per_s
            bt = r // nq
            q = r - bt * nq
            return s, bt, q

        def start_g(u, p):
            k_off = (u - u0) * _BQ
            pltpu.async_copy(
                table_hbm.at[idx_all.at[pl.ds(k_off, _BQ)]], srcs[p], sem_g[p]
            )

        def wait_g(p):
            pltpu.make_async_copy(
                table_hbm.at[pl.ds(0, _BQ)], srcs[p], sem_g[p]
            ).wait()

        def start_w(u, p):
            s, bt, q = decode(u)
            pltpu.async_copy(
                dsts[p],
                out_hbm.at[s, :, bt, :, pl.ds(q * _BQ, _BQ)],
                sem_w[p],
            )

        def wait_w(p):
            pltpu.make_async_copy(
                dsts[p], out_hbm.at[0, :, 0, :, pl.ds(0, _BQ)], sem_w[p]
            ).wait()

        rows = lax.iota(jnp.int32, 16)

        def transpose(p):
            src, dst = srcs[p], dsts[p]

            @plsc.parallel_loop(0, ndt, unroll=25)
            def body(dt):
                col0 = lax.broadcast(dt * 8, (16,))
                for jd in range(8):
                    v = plsc.load_gather(src, [rows, col0 + jd])
                    dst[dt, jd, pl.ds(0, 16)] = v

        for p in range(_NB - 1):
            start_g(u0 + p, p)

        for p in range(_NB):
            wait_g(p)
            start_g(u0 + p + 3, (p + 3) % _NB)
            transpose(p)
            start_w(u0 + p, p)

        def block(i, carry):
            u = u0 + i * _NB
            for p in range(_NB):
                wait_g(p)
                start_g(u + p + 3, (p + 3) % _NB)
                wait_w(p)
                transpose(p)
                start_w(u + p, p)
            return carry

        lax.fori_loop(1, upw // _NB - 1, block, 0)

        for p in range(_NB):
            u = u0 + upw - _NB + p
            wait_g(p)
            if p == 0:
                start_g(u + 3, (p + 3) % _NB)
            wait_w(p)
            transpose(p)
            start_w(u, p)

        for p in range(_NB):
            wait_w(p)

    out = k(table, xt)
    return out.transpose(2, 4, 0, 1, 3).reshape(bsz, seq, d)


# speedup vs baseline: 1.3999x; 1.3999x over previous
import functools

import jax
import jax.numpy as jnp
from jax import lax
from jax.experimental import pallas as pl
from jax.experimental.pallas import tpu as pltpu
from jax.experimental.pallas import tpu_sc as plsc

_NW = 32
_BQ = 32


def kernel(x, table):
    bsz, seq = x.shape
    vocab, d = table.shape
    ndt = d // 8
    nbt = bsz // 128
    nq = 128 // _BQ
    nunits = seq * nbt * nq
    upw = nunits // _NW
    units_per_s = nbt * nq

    xt = jnp.transpose(x).reshape(-1).astype(jnp.int32)

    mesh = plsc.VectorSubcoreMesh(core_axis_name="c", subcore_axis_name="s")

    @functools.partial(
        pl.kernel,
        mesh=mesh,
        out_type=jax.ShapeDtypeStruct((seq, ndt, nbt, 8, 128), jnp.float32),
        compiler_params=pltpu.CompilerParams(
            use_tc_tiling_on_sc=False, needs_layout_passes=False
        ),
        scratch_types=[
            [pltpu.VMEM((_BQ, d), jnp.float32) for _ in range(2)],
            [pltpu.VMEM((ndt, 8, _BQ), jnp.float32) for _ in range(2)],
            pltpu.VMEM((upw * _BQ,), jnp.int32),
            [pltpu.SemaphoreType.DMA for _ in range(2)],
            [pltpu.SemaphoreType.DMA for _ in range(2)],
        ],
    )
    def k(table_hbm, xt_hbm, out_hbm, srcs, dsts, idx_all, sem_g, sem_w):
        wid = lax.axis_index("s") * 2 + lax.axis_index("c")
        u0 = wid * upw
        pltpu.sync_copy(xt_hbm.at[pl.ds(u0 * _BQ, upw * _BQ)], idx_all)

        def decode(u):
            s = u // units_per_s
            r = u - s * units_per_s
            bt = r // nq
            q = r - bt * nq
            return s, bt, q

        def start_g(u, p):
            k_off = (u - u0) * _BQ
            pltpu.async_copy(
                table_hbm.at[idx_all.at[pl.ds(k_off, _BQ)]], srcs[p], sem_g[p]
            )

        def wait_g(p):
            pltpu.make_async_copy(
                table_hbm.at[pl.ds(0, _BQ)], srcs[p], sem_g[p]
            ).wait()

        def start_w(u, p):
            s, bt, q = decode(u)
            pltpu.async_copy(
                dsts[p],
                out_hbm.at[s, :, bt, :, pl.ds(q * _BQ, _BQ)],
                sem_w[p],
            )

        def wait_w(p):
            pltpu.make_async_copy(
                dsts[p], out_hbm.at[0, :, 0, :, pl.ds(0, _BQ)], sem_w[p]
            ).wait()

        rows_lo = lax.iota(jnp.int32, 16)
        rows_hi = rows_lo + 16

        def transpose(p):
            src, dst = srcs[p], dsts[p]

            @plsc.parallel_loop(0, ndt, unroll=25)
            def body(dt):
                col0 = lax.broadcast(dt * 8, (16,))
                for jd in range(8):
                    cols = col0 + jd
                    v0 = plsc.load_gather(src, [rows_lo, cols])
                    v1 = plsc.load_gather(src, [rows_hi, cols])
                    dst[dt, jd, pl.ds(0, 16)] = v0
                    dst[dt, jd, pl.ds(16, 16)] = v1

        start_g(u0, 0)
        start_g(u0 + 1, 1)

        for p in range(2):
            wait_g(p)
            transpose(p)
            start_w(u0 + p, p)
            start_g(u0 + p + 2, p)

        def block(i, carry):
            u = u0 + i * 2
            for p in range(2):
                wait_g(p)
                wait_w(p)
                transpose(p)
                start_w(u + p, p)
                start_g(u + p + 2, p)
            return carry

        lax.fori_loop(1, upw // 2 - 1, block, 0)

        for p in range(2):
            wait_g(p)
            wait_w(p)
            transpose(p)
            start_w(u0 + upw - 2 + p, p)

        for p in range(2):
            wait_w(p)

    out = k(table, xt)
    return out.transpose(2, 4, 0, 1, 3).reshape(bsz, seq, d)
